# baseline (speedup 1.0000x reference)
import jax
import jax.numpy as jnp
from jax import lax
from jax.experimental import pallas as pl
from jax.experimental.pallas import tpu as pltpu
from jax.experimental.pallas import tpu_sc as plsc

N = 10000
NP = 10240
D = 256
DH = 128
E = 160000
NC = 2
NS = 16
CH = 80
QC = 40
EP = NS * CH * 128
RPT = NP // NS
RB = 1024

f32 = jnp.float32


def _deg_body(src_hbm, dst_hbm, degs_hbm, degd_hbm, idx_v, ones_v, stage_v,
              hist_sh, sem_h):
    c = lax.axis_index("c")
    s = lax.axis_index("s")
    for k in range(8):
        ones_v[pl.ds(k * 16, 16)] = jnp.ones((16,), f32)

    def _z(i, carry):
        stage_v[pl.ds(i * 16, 16)] = jnp.zeros((16,), f32)
        return carry

    lax.fori_loop(0, RPT // 16, _z, 0)
    pltpu.sync_copy(stage_v, hist_sh.at[pl.ds(s * RPT, RPT)])

    @pl.when(c == 0)
    def _():
        pltpu.sync_copy(src_hbm.at[s], idx_v)

    @pl.when(c == 1)
    def _():
        pltpu.sync_copy(dst_hbm.at[s], idx_v)

    plsc.subcore_barrier()

    def _acc(j, carry):
        pltpu.async_copy(ones_v, hist_sh.at[idx_v.at[j]], sem_h, add=True)
        return carry

    lax.fori_loop(0, CH, _acc, 0)

    def _drain(j, carry):
        pltpu.make_async_copy(ones_v, hist_sh.at[idx_v.at[0]], sem_h).wait()
        return carry

    lax.fori_loop(0, CH, _drain, 0)
    plsc.subcore_barrier()
    pltpu.sync_copy(hist_sh.at[pl.ds(s * RPT, RPT)], stage_v)

    @pl.when(c == 0)
    def _():
        pltpu.sync_copy(stage_v, degs_hbm.at[pl.ds(s * RPT, RPT)])

    @pl.when(c == 1)
    def _():
        pltpu.sync_copy(stage_v, degd_hbm.at[pl.ds(s * RPT, RPT)])


_SC_KERNELS = {}


def _sc_mesh():
    return plsc.VectorSubcoreMesh(
        core_axis_name="c", subcore_axis_name="s", num_cores=NC,
        num_subcores=NS)


def _deg(src_t, dst_t):
    if "deg" not in _SC_KERNELS:
        _SC_KERNELS["deg"] = pl.kernel(
            _deg_body,
            out_type=[jax.ShapeDtypeStruct((NP,), f32),
                      jax.ShapeDtypeStruct((NP,), f32)],
            mesh=_sc_mesh(),
            scratch_types=[pltpu.VMEM((CH, 128), jnp.int32),
                           pltpu.VMEM((128,), f32),
                           pltpu.VMEM((RPT,), f32),
                           pltpu.VMEM_SHARED((NP,), f32),
                           pltpu.SemaphoreType.DMA],
        )
    return _SC_KERNELS["deg"](src_t, dst_t)


def _agg_body(y_hbm, src_hbm, dst_hbm, o_hbm, srcv, dstv, buf0, buf1, acc_sh,
              sem_g0, sem_g1, sem_s0, sem_s1):
    c = lax.axis_index("c")
    s = lax.axis_index("s")

    def _z(i, carry):
        buf0[i // 8, pl.ds((i % 8) * 16, 16)] = jnp.zeros((16,), f32)
        return carry

    lax.fori_loop(0, 128 * 8, _z, 0)
    for k in range(RPT // 128):
        pltpu.sync_copy(buf0, acc_sh.at[pl.ds(s * RPT + k * 128, 128)])
    plsc.subcore_barrier()

    def _pipe(tbl, out):
        def _half(q, carry):
            pltpu.sync_copy(src_hbm.at[s, pl.ds(q * QC, QC)], srcv)
            pltpu.sync_copy(dst_hbm.at[s, pl.ds(q * QC, QC)], dstv)
            pltpu.async_copy(tbl.at[srcv.at[0]], buf0, sem_g0)
            pltpu.async_copy(tbl.at[srcv.at[1]], buf1, sem_g1)

            def _pair(p, carry2):
                j0 = 2 * p
                pltpu.make_async_copy(tbl.at[srcv.at[j0]], buf0, sem_g0).wait()
                pltpu.async_copy(buf0, acc_sh.at[dstv.at[j0]], sem_s0,
                                 add=True)
                pltpu.make_async_copy(tbl.at[srcv.at[j0 + 1]], buf1,
                                      sem_g1).wait()
                pltpu.async_copy(buf1, acc_sh.at[dstv.at[j0 + 1]], sem_s1,
                                 add=True)

                @pl.when(p < QC // 2 - 1)
                def _():
                    pltpu.make_async_copy(buf0, acc_sh.at[dstv.at[j0]],
                                          sem_s0).wait()
                    pltpu.async_copy(tbl.at[srcv.at[j0 + 2]], buf0, sem_g0)
                    pltpu.make_async_copy(buf1, acc_sh.at[dstv.at[j0 + 1]],
                                          sem_s1).wait()
                    pltpu.async_copy(tbl.at[srcv.at[j0 + 3]], buf1, sem_g1)

                return carry2

            lax.fori_loop(0, QC // 2, _pair, 0)
            pltpu.make_async_copy(buf0, acc_sh.at[dstv.at[QC - 2]],
                                  sem_s0).wait()
            pltpu.make_async_copy(buf1, acc_sh.at[dstv.at[QC - 1]],
                                  sem_s1).wait()
            return carry

        lax.fori_loop(0, CH // QC, _half, 0)
        plsc.subcore_barrier()
        for k in range(RPT // 128):
            r0 = s * RPT + k * 128
            pltpu.sync_copy(acc_sh.at[pl.ds(r0, 128)], buf0)
            pltpu.sync_copy(buf0, out.at[pl.ds(r0, 128)])

    @pl.when(c == 0)
    def _():
        _pipe(y_hbm.at[0], o_hbm.at[0])

    @pl.when(c == 1)
    def _():
        _pipe(y_hbm.at[1], o_hbm.at[1])


def _agg(y, src_t, dst_t):
    if "agg" not in _SC_KERNELS:
        _SC_KERNELS["agg"] = pl.kernel(
            _agg_body,
            out_type=jax.ShapeDtypeStruct((NC, NP, DH), f32),
            mesh=_sc_mesh(),
            scratch_types=[pltpu.VMEM((QC, 128), jnp.int32),
                           pltpu.VMEM((QC, 128), jnp.int32),
                           pltpu.VMEM((128, DH), f32),
                           pltpu.VMEM((128, DH), f32),
                           pltpu.VMEM_SHARED((NP, DH), f32),
                           pltpu.SemaphoreType.DMA,
                           pltpu.SemaphoreType.DMA,
                           pltpu.SemaphoreType.DMA,
                           pltpu.SemaphoreType.DMA],
        )
    return _SC_KERNELS["agg"](y, src_t, dst_t)


def _tc_a_body(x_ref, wfc_ref, bfc_ref, degs_ref, w1_ref, out_ref):
    t = jnp.dot(x_ref[...], wfc_ref[...], preferred_element_type=f32)
    t = jnp.maximum(t + bfc_ref[...], 0.0)
    ns = lax.rsqrt(jnp.maximum(degs_ref[...], 1.0))
    y = jnp.dot(t * ns, w1_ref[...], preferred_element_type=f32)
    out_ref[0] = y[:, :DH]
    out_ref[1] = y[:, DH:]


_tc_a = pl.pallas_call(
    _tc_a_body,
    grid=(NP // RB,),
    in_specs=[pl.BlockSpec((RB, D), lambda i: (i, 0)),
              pl.BlockSpec((D, D), lambda i: (0, 0)),
              pl.BlockSpec((1, D), lambda i: (0, 0)),
              pl.BlockSpec((RB, 1), lambda i: (i, 0)),
              pl.BlockSpec((D, D), lambda i: (0, 0))],
    out_specs=pl.BlockSpec((NC, RB, DH), lambda i: (0, i, 0)),
    out_shape=jax.ShapeDtypeStruct((NC, NP, DH), f32),
)


def _tc_b_body(g_ref, degd_ref, degs_ref, b_ref, w_ref, out_ref):
    a = jnp.concatenate([g_ref[0], g_ref[1]], axis=1)
    nd = lax.rsqrt(jnp.maximum(degd_ref[...], 1.0))
    ns = lax.rsqrt(jnp.maximum(degs_ref[...], 1.0))
    h = jnp.maximum(a * nd + b_ref[...], 0.0)
    y = jnp.dot(h * ns, w_ref[...], preferred_element_type=f32)
    out_ref[0] = y[:, :DH]
    out_ref[1] = y[:, DH:]


_tc_b = pl.pallas_call(
    _tc_b_body,
    grid=(NP // RB,),
    in_specs=[pl.BlockSpec((NC, RB, DH), lambda i: (0, i, 0)),
              pl.BlockSpec((RB, 1), lambda i: (i, 0)),
              pl.BlockSpec((RB, 1), lambda i: (i, 0)),
              pl.BlockSpec((1, D), lambda i: (0, 0)),
              pl.BlockSpec((D, D), lambda i: (0, 0))],
    out_specs=pl.BlockSpec((NC, RB, DH), lambda i: (0, i, 0)),
    out_shape=jax.ShapeDtypeStruct((NC, NP, DH), f32),
)


def _tc_c_body(g_ref, degd_ref, b_ref, out_ref):
    a = jnp.concatenate([g_ref[0], g_ref[1]], axis=1)
    nd = lax.rsqrt(jnp.maximum(degd_ref[...], 1.0))
    out_ref[...] = a * nd + b_ref[...]


_tc_c = pl.pallas_call(
    _tc_c_body,
    grid=(NP // RB,),
    in_specs=[pl.BlockSpec((NC, RB, DH), lambda i: (0, i, 0)),
              pl.BlockSpec((RB, 1), lambda i: (i, 0)),
              pl.BlockSpec((1, D), lambda i: (0, 0))],
    out_specs=pl.BlockSpec((RB, D), lambda i: (i, 0)),
    out_shape=jax.ShapeDtypeStruct((NP, D), f32),
)


def kernel(x, edge_index, Wfc, bfc, W1, b1, W2, b2, W3, b3, W4, b4):
    pad = jnp.full((EP - E,), N, jnp.int32)
    src_t = jnp.concatenate([edge_index[0], pad]).reshape(NS, CH, 128)
    dst_t = jnp.concatenate([edge_index[1], pad]).reshape(NS, CH, 128)
    x_p = jnp.pad(x, ((0, NP - N), (0, 0)))

    degs, degd = _deg(src_t, dst_t)
    degs = degs.reshape(NP, 1)
    degd = degd.reshape(NP, 1)

    y = _tc_a(x_p, Wfc, bfc.reshape(1, D), degs, W1)
    g = _agg(y, src_t, dst_t)
    for b_prev, W in ((b1, W2), (b2, W3), (b3, W4)):
        y = _tc_b(g, degd, degs, b_prev.reshape(1, D), W)
        g = _agg(y, src_t, dst_t)
    out = _tc_c(g, degd, b4.reshape(1, D))
    return out[:N]


# speedup vs baseline: 1.0726x; 1.0726x over previous
import jax
import jax.numpy as jnp
from jax import lax
from jax.experimental import pallas as pl
from jax.experimental.pallas import tpu as pltpu
from jax.experimental.pallas import tpu_sc as plsc

N = 10000
NP = 10240
D = 256
DH = 128
E = 160000
NC = 2
NS = 16
CH = 80
QC = 40
EP = NS * CH * 128
RPT = NP // NS
RB = 1024

f32 = jnp.float32


def _deg_body(src_hbm, dst_hbm, degs_hbm, degd_hbm, idx_v, ones_v, stage_v,
              hist_sh):
    c = lax.axis_index("c")
    s = lax.axis_index("s")
    for k in range(8):
        ones_v[pl.ds(k * 16, 16)] = jnp.ones((16,), f32)

    def _z(i, carry):
        stage_v[pl.ds(i * 16, 16)] = jnp.zeros((16,), f32)
        return carry

    lax.fori_loop(0, RPT // 16, _z, 0)
    pltpu.sync_copy(stage_v, hist_sh.at[pl.ds(s * RPT, RPT)])

    @pl.when(c == 0)
    def _():
        pltpu.sync_copy(src_hbm.at[s], idx_v)

    @pl.when(c == 1)
    def _():
        pltpu.sync_copy(dst_hbm.at[s], idx_v)

    plsc.subcore_barrier()

    def _acc(j, carry):
        pltpu.sync_copy(ones_v, hist_sh.at[idx_v.at[j]], add=True)
        return carry

    lax.fori_loop(0, CH, _acc, 0)
    plsc.subcore_barrier()
    pltpu.sync_copy(hist_sh.at[pl.ds(s * RPT, RPT)], stage_v)

    @pl.when(c == 0)
    def _():
        pltpu.sync_copy(stage_v, degs_hbm.at[pl.ds(s * RPT, RPT)])

    @pl.when(c == 1)
    def _():
        pltpu.sync_copy(stage_v, degd_hbm.at[pl.ds(s * RPT, RPT)])


_SC_KERNELS = {}


def _sc_mesh():
    return plsc.VectorSubcoreMesh(
        core_axis_name="c", subcore_axis_name="s", num_cores=NC,
        num_subcores=NS)


def _deg(src_t, dst_t):
    if "deg" not in _SC_KERNELS:
        _SC_KERNELS["deg"] = pl.kernel(
            _deg_body,
            out_type=[jax.ShapeDtypeStruct((NP,), f32),
                      jax.ShapeDtypeStruct((NP,), f32)],
            mesh=_sc_mesh(),
            scratch_types=[pltpu.VMEM((CH, 128), jnp.int32),
                           pltpu.VMEM((128,), f32),
                           pltpu.VMEM((RPT,), f32),
                           pltpu.VMEM_SHARED((NP,), f32)],
        )
    return _SC_KERNELS["deg"](src_t, dst_t)


def _agg_body(y_hbm, src_hbm, dst_hbm, o_hbm, srcv, dstv, buf0, buf1, acc_sh,
              sem0, sem1):
    c = lax.axis_index("c")
    s = lax.axis_index("s")

    def _z(i, carry):
        buf0[i // 8, pl.ds((i % 8) * 16, 16)] = jnp.zeros((16,), f32)
        return carry

    lax.fori_loop(0, 128 * 8, _z, 0)
    for k in range(RPT // 128):
        pltpu.sync_copy(buf0, acc_sh.at[pl.ds(s * RPT + k * 128, 128)])
    plsc.subcore_barrier()

    def _pipe(tbl, out):
        def _half(q, carry):
            pltpu.sync_copy(src_hbm.at[s, pl.ds(q * QC, QC)], srcv)
            pltpu.sync_copy(dst_hbm.at[s, pl.ds(q * QC, QC)], dstv)
            pltpu.async_copy(tbl.at[srcv.at[0]], buf0, sem0)

            def _pair(p, carry2):
                j0 = 2 * p
                pltpu.async_copy(tbl.at[srcv.at[j0 + 1]], buf1, sem1)
                pltpu.make_async_copy(tbl.at[srcv.at[j0]], buf0, sem0).wait()
                pltpu.sync_copy(buf0, acc_sh.at[dstv.at[j0]], add=True)

                @pl.when(p < QC // 2 - 1)
                def _():
                    pltpu.async_copy(tbl.at[srcv.at[j0 + 2]], buf0, sem0)

                pltpu.make_async_copy(tbl.at[srcv.at[j0 + 1]], buf1,
                                      sem1).wait()
                pltpu.sync_copy(buf1, acc_sh.at[dstv.at[j0 + 1]], add=True)
                return carry2

            lax.fori_loop(0, QC // 2, _pair, 0)
            return carry

        lax.fori_loop(0, CH // QC, _half, 0)
        plsc.subcore_barrier()
        for k in range(RPT // 128):
            r0 = s * RPT + k * 128
            pltpu.sync_copy(acc_sh.at[pl.ds(r0, 128)], buf0)
            pltpu.sync_copy(buf0, out.at[pl.ds(r0, 128)])

    @pl.when(c == 0)
    def _():
        _pipe(y_hbm.at[0], o_hbm.at[0])

    @pl.when(c == 1)
    def _():
        _pipe(y_hbm.at[1], o_hbm.at[1])


def _agg(y, src_t, dst_t):
    if "agg" not in _SC_KERNELS:
        _SC_KERNELS["agg"] = pl.kernel(
            _agg_body,
            out_type=jax.ShapeDtypeStruct((NC, NP, DH), f32),
            mesh=_sc_mesh(),
            scratch_types=[pltpu.VMEM((QC, 128), jnp.int32),
                           pltpu.VMEM((QC, 128), jnp.int32),
                           pltpu.VMEM((128, DH), f32),
                           pltpu.VMEM((128, DH), f32),
                           pltpu.VMEM_SHARED((NP, DH), f32),
                           pltpu.SemaphoreType.DMA,
                           pltpu.SemaphoreType.DMA],
        )
    return _SC_KERNELS["agg"](y, src_t, dst_t)


def _tc_a_body(x_ref, wfc_ref, bfc_ref, degs_ref, w1_ref, out_ref):
    t = jnp.dot(x_ref[...], wfc_ref[...], preferred_element_type=f32)
    t = jnp.maximum(t + bfc_ref[...], 0.0)
    ns = lax.rsqrt(jnp.maximum(degs_ref[...], 1.0))
    y = jnp.dot(t * ns, w1_ref[...], preferred_element_type=f32)
    out_ref[0] = y[:, :DH]
    out_ref[1] = y[:, DH:]


_tc_a = pl.pallas_call(
    _tc_a_body,
    grid=(NP // RB,),
    in_specs=[pl.BlockSpec((RB, D), lambda i: (i, 0)),
              pl.BlockSpec((D, D), lambda i: (0, 0)),
              pl.BlockSpec((1, D), lambda i: (0, 0)),
              pl.BlockSpec((RB, 1), lambda i: (i, 0)),
              pl.BlockSpec((D, D), lambda i: (0, 0))],
    out_specs=pl.BlockSpec((NC, RB, DH), lambda i: (0, i, 0)),
    out_shape=jax.ShapeDtypeStruct((NC, NP, DH), f32),
)


def _tc_b_body(g_ref, degd_ref, degs_ref, b_ref, w_ref, out_ref):
    a = jnp.concatenate([g_ref[0], g_ref[1]], axis=1)
    nd = lax.rsqrt(jnp.maximum(degd_ref[...], 1.0))
    ns = lax.rsqrt(jnp.maximum(degs_ref[...], 1.0))
    h = jnp.maximum(a * nd + b_ref[...], 0.0)
    y = jnp.dot(h * ns, w_ref[...], preferred_element_type=f32)
    out_ref[0] = y[:, :DH]
    out_ref[1] = y[:, DH:]


_tc_b = pl.pallas_call(
    _tc_b_body,
    grid=(NP // RB,),
    in_specs=[pl.BlockSpec((NC, RB, DH), lambda i: (0, i, 0)),
              pl.BlockSpec((RB, 1), lambda i: (i, 0)),
              pl.BlockSpec((RB, 1), lambda i: (i, 0)),
              pl.BlockSpec((1, D), lambda i: (0, 0)),
              pl.BlockSpec((D, D), lambda i: (0, 0))],
    out_specs=pl.BlockSpec((NC, RB, DH), lambda i: (0, i, 0)),
    out_shape=jax.ShapeDtypeStruct((NC, NP, DH), f32),
)


def _tc_c_body(g_ref, degd_ref, b_ref, out_ref):
    a = jnp.concatenate([g_ref[0], g_ref[1]], axis=1)
    nd = lax.rsqrt(jnp.maximum(degd_ref[...], 1.0))
    out_ref[...] = a * nd + b_ref[...]


_tc_c = pl.pallas_call(
    _tc_c_body,
    grid=(NP // RB,),
    in_specs=[pl.BlockSpec((NC, RB, DH), lambda i: (0, i, 0)),
              pl.BlockSpec((RB, 1), lambda i: (i, 0)),
              pl.BlockSpec((1, D), lambda i: (0, 0))],
    out_specs=pl.BlockSpec((RB, D), lambda i: (i, 0)),
    out_shape=jax.ShapeDtypeStruct((NP, D), f32),
)


def kernel(x, edge_index, Wfc, bfc, W1, b1, W2, b2, W3, b3, W4, b4):
    pad = jnp.full((EP - E,), N, jnp.int32)
    src_t = jnp.concatenate([edge_index[0], pad]).reshape(NS, CH, 128)
    dst_t = jnp.concatenate([edge_index[1], pad]).reshape(NS, CH, 128)
    x_p = jnp.pad(x, ((0, NP - N), (0, 0)))

    degs, degd = _deg(src_t, dst_t)
    degs = degs.reshape(NP, 1)
    degd = degd.reshape(NP, 1)

    y = _tc_a(x_p, Wfc, bfc.reshape(1, D), degs, W1)
    g = _agg(y, src_t, dst_t)
    for b_prev, W in ((b1, W2), (b2, W3), (b3, W4)):
        y = _tc_b(g, degd, degs, b_prev.reshape(1, D), W)
        g = _agg(y, src_t, dst_t)
    out = _tc_c(g, degd, b4.reshape(1, D))
    return out[:N]
